# Initial kernel scaffold; baseline (speedup 1.0000x reference)
import jax
import jax.numpy as jnp
from jax.experimental import pallas as pl


def kernel(input, embeddings, step):
    raise NotImplementedError("write your pallas kernel here")


# speedup vs baseline: 1.0666x; 1.0666x over previous
import functools
import math

import jax
import jax.numpy as jnp
from jax import lax
from jax.experimental import pallas as pl
from jax.experimental.pallas import tpu as pltpu
from jax.experimental.pallas import tpu_sc as plsc

_MAX_SEQ = 2048
_PAD = 0
_NUM_CORES = 2
_NUM_SUBCORES = 16
_LANES = 16


def _pos_table(num_pos, dim):
    half = dim // 2
    e = math.log(10000.0) / (half - 1)
    e = jnp.exp(jnp.arange(half, dtype=jnp.float32) * -e)
    pe = jnp.arange(num_pos, dtype=jnp.float32)[:, None] * e[None, :]
    pe = jnp.concatenate([jnp.sin(pe), jnp.cos(pe)], axis=1).reshape(num_pos, -1)
    if dim % 2 == 1:
        pe = jnp.concatenate([pe, jnp.zeros((num_pos, 1), dtype=jnp.float32)], axis=1)
    return pe


def _make_sc_kernel(B, D, chunk, sl, scale):
    n_idx = chunk // 128
    mesh = plsc.VectorSubcoreMesh(core_axis_name="c", subcore_axis_name="s")

    @functools.partial(
        pl.kernel,
        mesh=mesh,
        out_type=jax.ShapeDtypeStruct((B, D), jnp.float32),
        scratch_types=[
            pltpu.VMEM((n_idx, 128), jnp.int32),
            pltpu.VMEM((n_idx, 128), jnp.int32),
            pltpu.VMEM((chunk, D), jnp.float32),
            pltpu.VMEM((chunk, D), jnp.float32),
            pltpu.SemaphoreType.DMA,
        ],
    )
    def k(tok_hbm, posx_hbm, emb_hbm, out_hbm, tokv, pidxv, rows, posr, sem):
        wid = lax.axis_index("s") * _NUM_CORES + lax.axis_index("c")
        base = wid * chunk
        p0 = lax.rem(base, sl)

        for j in range(n_idx):
            pltpu.sync_copy(tok_hbm.at[pl.ds(base + j * 128, 128)], tokv.at[j])

        for j in range(n_idx):
            for i in range(128 // _LANES):
                sli = pl.ds(i * _LANES, _LANES)
                t = tokv[j, sli]
                pv = lax.iota(jnp.int32, _LANES) + (p0 + j * 128 + i * _LANES)
                pidxv[j, sli] = jnp.where(t != _PAD, pv, sl)

        copies = []
        for j in range(n_idx):
            copies.append(pltpu.async_copy(
                emb_hbm.at[tokv.at[j]], rows.at[pl.ds(j * 128, 128)], sem))
            copies.append(pltpu.async_copy(
                posx_hbm.at[pidxv.at[j]], posr.at[pl.ds(j * 128, 128)], sem))
        for cp in copies:
            cp.wait()

        def body(r, carry):
            for i in range(D // _LANES):
                sli = pl.ds(i * _LANES, _LANES)
                rows[r, sli] = rows[r, sli] * scale + posr[r, sli]
            return carry

        lax.fori_loop(0, chunk, body, 0)
        pltpu.sync_copy(rows, out_hbm.at[pl.ds(base, chunk)])

    return k


def kernel(input, embeddings, step):
    bs, sl = input.shape
    dim = embeddings.shape[1]
    B = bs * sl
    scale = float(dim) ** 0.5
    pos = _pos_table(_MAX_SEQ, dim)
    pos_slice = lax.dynamic_slice_in_dim(pos, step, sl, axis=0)
    posx = jnp.concatenate([pos_slice, jnp.zeros((1, dim), jnp.float32)], axis=0)
    tok = input.reshape(-1)
    chunk = B // (_NUM_CORES * _NUM_SUBCORES)
    k = _make_sc_kernel(B, dim, chunk, sl, scale)
    out = k(tok, posx, embeddings)
    return out.reshape(bs, sl, dim)


# speedup vs baseline: 1.1196x; 1.0497x over previous
import functools
import math

import jax
import jax.numpy as jnp
from jax import lax
from jax.experimental import pallas as pl
from jax.experimental.pallas import tpu as pltpu
from jax.experimental.pallas import tpu_sc as plsc

_MAX_SEQ = 2048
_PAD = 0
_NUM_CORES = 2
_NUM_SUBCORES = 16
_LANES = 16


def _pos_table(num_pos, dim):
    half = dim // 2
    e = math.log(10000.0) / (half - 1)
    e = jnp.exp(jnp.arange(half, dtype=jnp.float32) * -e)
    pe = jnp.arange(num_pos, dtype=jnp.float32)[:, None] * e[None, :]
    pe = jnp.concatenate([jnp.sin(pe), jnp.cos(pe)], axis=1).reshape(num_pos, -1)
    if dim % 2 == 1:
        pe = jnp.concatenate([pe, jnp.zeros((num_pos, 1), dtype=jnp.float32)], axis=1)
    return pe


def _make_sc_kernel(B, D, chunk, sl, scale):
    NB = 4
    BR = chunk // NB
    mesh = plsc.VectorSubcoreMesh(core_axis_name="c", subcore_axis_name="s")

    @functools.partial(
        pl.kernel,
        mesh=mesh,
        out_type=jax.ShapeDtypeStruct((B, D), jnp.float32),
        scratch_types=[
            pltpu.VMEM((NB, BR), jnp.int32),
            pltpu.VMEM((NB, BR), jnp.int32),
            pltpu.VMEM((chunk, D), jnp.float32),
            pltpu.VMEM((chunk, D), jnp.float32),
            pltpu.SemaphoreType.DMA,
            pltpu.SemaphoreType.DMA,
            pltpu.SemaphoreType.DMA,
            pltpu.SemaphoreType.DMA,
            pltpu.SemaphoreType.DMA,
            pltpu.SemaphoreType.DMA,
        ],
    )
    def k(tok_hbm, posx_hbm, emb_hbm, out_hbm, tokv, pidxv, rows, posr,
          sem_i, g0, g1, g2, g3, sem_o):
        gsems = [g0, g1, g2, g3]
        wid = lax.axis_index("s") * _NUM_CORES + lax.axis_index("c")
        base = wid * chunk
        p0 = lax.rem(base, sl)

        idx_cps = [
            pltpu.async_copy(tok_hbm.at[pl.ds(base + b * BR, BR)],
                             tokv.at[b], sem_i)
            for b in range(NB)
        ]
        for cp in idx_cps:
            cp.wait()

        gather_cps = []
        for b in range(NB):
            for i in range(BR // _LANES):
                sli = pl.ds(i * _LANES, _LANES)
                t = tokv[b, sli]
                pv = lax.iota(jnp.int32, _LANES) + (p0 + b * BR + i * _LANES)
                pidxv[b, sli] = jnp.where(t != _PAD, pv, sl)
            gather_cps.append((
                pltpu.async_copy(emb_hbm.at[tokv.at[b]],
                                 rows.at[pl.ds(b * BR, BR)], gsems[b]),
                pltpu.async_copy(posx_hbm.at[pidxv.at[b]],
                                 posr.at[pl.ds(b * BR, BR)], gsems[b]),
            ))

        def body(r, carry):
            for i in range(D // _LANES):
                sli = pl.ds(i * _LANES, _LANES)
                rows[r, sli] = rows[r, sli] * scale + posr[r, sli]
            return carry

        store_cps = []
        for b in range(NB):
            cp_e, cp_p = gather_cps[b]
            cp_e.wait()
            cp_p.wait()
            lax.fori_loop(b * BR, (b + 1) * BR, body, 0)
            store_cps.append(pltpu.async_copy(
                rows.at[pl.ds(b * BR, BR)],
                out_hbm.at[pl.ds(base + b * BR, BR)], sem_o))
        for cp in store_cps:
            cp.wait()

    return k


def kernel(input, embeddings, step):
    bs, sl = input.shape
    dim = embeddings.shape[1]
    B = bs * sl
    scale = float(dim) ** 0.5
    pos = _pos_table(_MAX_SEQ, dim)
    pos_slice = lax.dynamic_slice_in_dim(pos, step, sl, axis=0)
    posx = jnp.concatenate([pos_slice, jnp.zeros((1, dim), jnp.float32)], axis=0)
    tok = input.reshape(-1)
    chunk = B // (_NUM_CORES * _NUM_SUBCORES)
    k = _make_sc_kernel(B, dim, chunk, sl, scale)
    out = k(tok, posx, embeddings)
    return out.reshape(bs, sl, dim)
